# baseline (speedup 1.0000x reference)
import jax
import jax.numpy as jnp
from jax import lax
from jax.experimental import pallas as pl
from jax.experimental.pallas import tpu as pltpu
from jax.experimental.pallas import tpu_sc as plsc

_N = 10000
_E = 320000
_D = 128
_EPS = 1e-5

_MP = 10240
_CHUNK = 128
_NCH = _E // _CHUNK
_NCHP = 2560
_WCH = _NCHP // 32
_DUMP = 10232
_RPT = _MP // 16
_CW = 16


_MESH = plsc.VectorSubcoreMesh(core_axis_name="c", subcore_axis_name="s")


def _sc_agg():
    def body(src, gidx, sidx, zrow, ones_h,
             agg_o, cnt_o, gbuf, sbuf, rows0, rows1, gsem, ssem, acc):
        cid = lax.axis_index("c")
        sid = lax.axis_index("s")
        wid = sid * 2 + cid
        base = wid * _WCH
        r0 = sid * _RPT
        half = _WCH // 2

        def zero_acc():
            pltpu.sync_copy(zrow.at[pl.ds(r0, _RPT)], acc.at[pl.ds(r0, _RPT)])

        pltpu.sync_copy(ones_h, rows0)
        zero_acc()
        plsc.subcore_barrier()

        def cfire(j, carry):
            pltpu.async_copy(rows0, acc.at[sbuf.at[carry + j]], ssem,
                             add=True)
            return carry

        for h in range(2):
            pltpu.sync_copy(sidx.at[pl.ds(base + h * half, half)], sbuf)
            for b in range(half // 8):
                lax.fori_loop(0, 8, cfire, b * 8)
                for _ in range(8):
                    pltpu.make_async_copy(zrow.at[pl.ds(0, _CHUNK)], rows1,
                                          ssem).wait()
        plsc.subcore_barrier()
        off = cid * _MP + r0
        pltpu.sync_copy(acc.at[pl.ds(r0, _RPT)], cnt_o.at[pl.ds(off, _RPT)])
        plsc.subcore_barrier()

        zero_acc()
        pltpu.sync_copy(gidx.at[pl.ds(base, _WCH)], gbuf)
        plsc.subcore_barrier()

        def wait_g(buf):
            pltpu.make_async_copy(src.at[gbuf.at[0]], buf, gsem).wait()

        def wait_s():
            pltpu.make_async_copy(zrow.at[pl.ds(0, _CHUNK)], rows1,
                                  ssem).wait()

        def scat(local_j, buf):
            pltpu.async_copy(buf, acc.at[sbuf.at[local_j]], ssem, add=True)

        def bstep(g, h):
            j0 = h * half + 2 * g
            hi = h * half + half - 1
            wait_g(rows0)
            scat(2 * g, rows0)
            wait_s()
            pltpu.async_copy(src.at[gbuf.at[j0 + 1]], rows1, gsem)
            wait_g(rows1)
            scat(2 * g + 1, rows1)
            wait_s()
            j2 = jnp.minimum(j0 + 2, hi)
            pltpu.async_copy(src.at[gbuf.at[j2]], rows0, gsem)
            return h

        for h in range(2):
            c0 = h * half
            pltpu.sync_copy(sidx.at[pl.ds(base + c0, half)], sbuf)
            pltpu.async_copy(src.at[gbuf.at[c0]], rows0, gsem)
            wait_g(rows0)
            scat(0, rows0)
            pltpu.async_copy(src.at[gbuf.at[c0 + 1]], rows1, gsem)
            wait_g(rows1)
            scat(1, rows1)
            wait_s()
            pltpu.async_copy(src.at[gbuf.at[c0 + 2]], rows0, gsem)
            lax.fori_loop(1, half // 2, bstep, h)
            wait_s()
            wait_g(rows0)
        plsc.subcore_barrier()
        pltpu.sync_copy(acc.at[pl.ds(r0, _RPT)], agg_o.at[pl.ds(off, _RPT)])

    return pl.kernel(
        body,
        out_type=[jax.ShapeDtypeStruct((2 * _MP, _D), jnp.float32),
                  jax.ShapeDtypeStruct((2 * _MP, _D), jnp.float32)],
        mesh=_MESH,
        scratch_types=[
            pltpu.VMEM((_WCH, _CHUNK), jnp.int32),
            pltpu.VMEM((_WCH // 2, _CHUNK), jnp.int32),
            pltpu.VMEM((_CHUNK, _D), jnp.float32),
            pltpu.VMEM((_CHUNK, _D), jnp.float32),
            pltpu.SemaphoreType.DMA,
            pltpu.SemaphoreType.DMA,
            pltpu.VMEM_SHARED((_MP, _D), jnp.float32),
        ])


_BLK = 640
_NB = _MP // _BLK


def _full(shape):
    return pl.BlockSpec(shape, lambda i: tuple(0 for _ in shape))


def _ln(x, g, b):
    mu = jnp.mean(x, axis=1, keepdims=True)
    xc = x - mu
    var = jnp.mean(xc * xc, axis=1, keepdims=True)
    return xc * lax.rsqrt(var + _EPS) * g + b


def _tc_pre(x, g, b, wt, bias):
    def body(x_ref, g_ref, b_ref, wt_ref, bb_ref, o_ref):
        xn = _ln(x_ref[...], g_ref[...], b_ref[...])
        o_ref[...] = (jnp.dot(xn, wt_ref[...], preferred_element_type=jnp.float32)
                      + bb_ref[...])
    return pl.pallas_call(
        body,
        grid=(_NB,),
        in_specs=[pl.BlockSpec((_BLK, _D), lambda i: (i, 0)),
                  _full((1, _D)), _full((1, _D)), _full((_D, _D)), _full((1, _D))],
        out_specs=pl.BlockSpec((_BLK, _D), lambda i: (i, 0)),
        out_shape=jax.ShapeDtypeStruct((_MP, _D), jnp.float32),
    )(x, g.reshape(1, _D), b.reshape(1, _D), wt, bias.reshape(1, _D))


def _tc_edge(aggp, cntp, wem_t, bem, g_e, b_e, we_t, be):
    def body(a0, a1, c0, c1, wem_ref, bem_ref, g_ref, b_ref, wet_ref, be_ref, o_ref):
        s = a0[...] + a1[...]
        c = (c0[...] + c1[...])[:, 0:1]
        ya = s / jnp.clip(c, 1.0, None)
        y = (jnp.dot(ya, wem_ref[...], preferred_element_type=jnp.float32)
             + bem_ref[...])
        y = _ln(y, g_ref[...], b_ref[...])
        o_ref[...] = (jnp.dot(y, wet_ref[...], preferred_element_type=jnp.float32)
                      + be_ref[...])
    return pl.pallas_call(
        body,
        grid=(_NB,),
        in_specs=[pl.BlockSpec((_BLK, _D), lambda i: (i, 0)),
                  pl.BlockSpec((_BLK, _D), lambda i: (i + _NB, 0)),
                  pl.BlockSpec((_BLK, _D), lambda i: (i, 0)),
                  pl.BlockSpec((_BLK, _D), lambda i: (i + _NB, 0)),
                  _full((_D, _D)), _full((1, _D)), _full((1, _D)), _full((1, _D)),
                  _full((_D, _D)), _full((1, _D))],
        out_specs=pl.BlockSpec((_BLK, _D), lambda i: (i, 0)),
        out_shape=jax.ShapeDtypeStruct((_MP, _D), jnp.float32),
    )(aggp, aggp, cntp, cntp, wem_t, bem.reshape(1, _D),
      g_e.reshape(1, _D), b_e.reshape(1, _D), we_t, be.reshape(1, _D))


def _tc_post(zp, cntp, x0, w1t, w2t, bvm, g, b):
    def body(z0, z1, c0, c1, x_ref, w1_ref, w2_ref, bb_ref, g_ref, b_ref, o_ref):
        s = z0[...] + z1[...]
        c = (c0[...] + c1[...])[:, 0:1]
        z = s / jnp.clip(c, 1.0, None)
        x = x_ref[...]
        h = (jnp.dot(x, w1_ref[...], preferred_element_type=jnp.float32)
             + jnp.dot(z, w2_ref[...], preferred_element_type=jnp.float32)
             + bb_ref[...])
        o_ref[...] = _ln(h, g_ref[...], b_ref[...]) + x
    return pl.pallas_call(
        body,
        grid=(_NB,),
        in_specs=[pl.BlockSpec((_BLK, _D), lambda i: (i, 0)),
                  pl.BlockSpec((_BLK, _D), lambda i: (i + _NB, 0)),
                  pl.BlockSpec((_BLK, _D), lambda i: (i, 0)),
                  pl.BlockSpec((_BLK, _D), lambda i: (i + _NB, 0)),
                  pl.BlockSpec((_BLK, _D), lambda i: (i, 0)),
                  _full((_D, _D)), _full((_D, _D)), _full((1, _D)),
                  _full((1, _D)), _full((1, _D))],
        out_specs=pl.BlockSpec((_BLK, _D), lambda i: (i, 0)),
        out_shape=jax.ShapeDtypeStruct((_MP, _D), jnp.float32),
    )(zp, zp, cntp, cntp, x0, w1t, w2t, bvm.reshape(1, _D),
      g.reshape(1, _D), b.reshape(1, _D))


@jax.jit
def kernel(X, edge_index, Wv, bv, Wem, bem, We, be, Wvm, bvm,
           g_pre, b_pre, g_e, b_e, g_post, b_post):
    idx_pad = ((0, _NCHP - _NCH), (0, 0))
    v_idx = jnp.pad(edge_index[0].astype(jnp.int32).reshape(_NCH, _CHUNK),
                    idx_pad, constant_values=_DUMP)
    e_idx = jnp.pad(edge_index[1].astype(jnp.int32).reshape(_NCH, _CHUNK),
                    idx_pad, constant_values=_DUMP)
    x_pad = jnp.pad(X, ((0, _MP - _N), (0, 0)))
    zrow = jnp.zeros((_MP, _D), jnp.float32)
    ones = jnp.ones((_CHUNK, _D), jnp.float32)

    Xv = _tc_pre(x_pad, g_pre, b_pre, Wv.T, bv)
    aggp, cep = _sc_agg()(Xv, v_idx, e_idx, zrow, ones)
    Y = _tc_edge(aggp, cep, Wem.T, bem, g_e, b_e, We.T, be)
    zp, cvp = _sc_agg()(Y, e_idx, v_idx, zrow, ones)
    out = _tc_post(zp, cvp, x_pad, Wvm[:, :_D].T, Wvm[:, _D:].T, bvm,
                   g_post, b_post)
    return out[:_N]


# speedup vs baseline: 2.3344x; 2.3344x over previous
import jax
import jax.numpy as jnp
from jax import lax
from jax.experimental import pallas as pl
from jax.experimental.pallas import tpu as pltpu
from jax.experimental.pallas import tpu_sc as plsc

_N = 10000
_E = 320000
_D = 128
_EPS = 1e-5

_MP = 10240
_CHUNK = 128
_NCH = _E // _CHUNK
_NCHP = 2560
_WCH = _NCHP // 32
_DUMP = 10232
_RPT = _MP // 16
_CW = 16


_MESH = plsc.VectorSubcoreMesh(core_axis_name="c", subcore_axis_name="s")


def _sc_agg():
    def body(src, gidx, sidx, zrow, ones_h,
             agg_o, cnt_o, gbuf, sbuf, rows0, rows1, gsem, ssem, acc):
        cid = lax.axis_index("c")
        sid = lax.axis_index("s")
        wid = sid * 2 + cid
        base = wid * _WCH
        r0 = sid * _RPT
        half = _WCH // 2

        def zero_acc():
            pltpu.sync_copy(zrow.at[pl.ds(r0, _RPT)], acc.at[pl.ds(r0, _RPT)])

        pltpu.sync_copy(ones_h, rows0)
        zero_acc()
        plsc.subcore_barrier()

        def cfire(j, carry):
            pltpu.async_copy(rows0, acc.at[sbuf.at[carry + j]], ssem,
                             add=True)
            return carry

        for h in range(2):
            pltpu.sync_copy(sidx.at[pl.ds(base + h * half, half)], sbuf)
            for b in range(half // 8):
                lax.fori_loop(0, 8, cfire, b * 8)
                for _ in range(8):
                    pltpu.make_async_copy(zrow.at[pl.ds(0, _CHUNK)], rows1,
                                          ssem).wait()
        plsc.subcore_barrier()
        off = cid * _MP + r0
        pltpu.sync_copy(acc.at[pl.ds(r0, _RPT)], cnt_o.at[pl.ds(off, _RPT)])
        plsc.subcore_barrier()

        zero_acc()
        pltpu.sync_copy(gidx.at[pl.ds(base, _WCH)], gbuf)
        plsc.subcore_barrier()

        def wait_g(buf):
            pltpu.make_async_copy(src.at[gbuf.at[0]], buf, gsem).wait()

        def wait_s():
            pltpu.make_async_copy(zrow.at[pl.ds(0, _CHUNK)], rows1,
                                  ssem).wait()

        def scat(local_j, buf):
            pltpu.async_copy(buf, acc.at[sbuf.at[local_j]], ssem, add=True)

        def bstep(g, h):
            j0 = h * half + 2 * g
            hi = h * half + half - 1
            wait_g(rows0)
            scat(2 * g, rows0)
            wait_s()
            pltpu.async_copy(src.at[gbuf.at[j0 + 1]], rows1, gsem)
            wait_g(rows1)
            scat(2 * g + 1, rows1)
            wait_s()
            j2 = jnp.minimum(j0 + 2, hi)
            pltpu.async_copy(src.at[gbuf.at[j2]], rows0, gsem)
            return h

        for h in range(2):
            c0 = h * half
            pltpu.sync_copy(sidx.at[pl.ds(base + c0, half)], sbuf)
            pltpu.async_copy(src.at[gbuf.at[c0]], rows0, gsem)
            wait_g(rows0)
            scat(0, rows0)
            pltpu.async_copy(src.at[gbuf.at[c0 + 1]], rows1, gsem)
            wait_g(rows1)
            scat(1, rows1)
            wait_s()
            pltpu.async_copy(src.at[gbuf.at[c0 + 2]], rows0, gsem)
            lax.fori_loop(1, half // 2, bstep, h)
            wait_s()
            wait_g(rows0)
        plsc.subcore_barrier()
        pltpu.sync_copy(acc.at[pl.ds(r0, _RPT)], agg_o.at[pl.ds(off, _RPT)])

    return pl.kernel(
        body,
        out_type=[jax.ShapeDtypeStruct((2 * _MP, _D), jnp.float32),
                  jax.ShapeDtypeStruct((2 * _MP, _D), jnp.float32)],
        mesh=_MESH,
        scratch_types=[
            pltpu.VMEM((_WCH, _CHUNK), jnp.int32),
            pltpu.VMEM((_WCH // 2, _CHUNK), jnp.int32),
            pltpu.VMEM((_CHUNK, _D), jnp.float32),
            pltpu.VMEM((_CHUNK, _D), jnp.float32),
            pltpu.SemaphoreType.DMA,
            pltpu.SemaphoreType.DMA,
            pltpu.VMEM_SHARED((_MP, _D), jnp.float32),
        ])


_BLK = 640
_NB = _MP // _BLK


def _full(shape):
    return pl.BlockSpec(shape, lambda i: tuple(0 for _ in shape))


def _ln(x, g, b):
    mu = jnp.mean(x, axis=1, keepdims=True)
    xc = x - mu
    var = jnp.mean(xc * xc, axis=1, keepdims=True)
    return xc * lax.rsqrt(var + _EPS) * g + b


def _tc_pre(x, g, b, wt, bias):
    def body(x_ref, g_ref, b_ref, wt_ref, bb_ref, o_ref):
        xn = _ln(x_ref[...], g_ref[...], b_ref[...])
        o_ref[...] = (jnp.dot(xn, wt_ref[...], preferred_element_type=jnp.float32)
                      + bb_ref[...])
    return pl.pallas_call(
        body,
        grid=(_NB,),
        in_specs=[pl.BlockSpec((_BLK, _D), lambda i: (i, 0)),
                  _full((1, _D)), _full((1, _D)), _full((_D, _D)), _full((1, _D))],
        out_specs=pl.BlockSpec((_BLK, _D), lambda i: (i, 0)),
        out_shape=jax.ShapeDtypeStruct((_MP, _D), jnp.float32),
    )(x, g.reshape(1, _D), b.reshape(1, _D), wt, bias.reshape(1, _D))


def _tc_edge(aggp, cntp, wem_t, bem, g_e, b_e, we_t, be):
    def body(a0, a1, c0, c1, wem_ref, bem_ref, g_ref, b_ref, wet_ref, be_ref, o_ref):
        s = a0[...] + a1[...]
        c = (c0[...] + c1[...])[:, 0:1]
        ya = s / jnp.clip(c, 1.0, None)
        y = (jnp.dot(ya, wem_ref[...], preferred_element_type=jnp.float32)
             + bem_ref[...])
        y = _ln(y, g_ref[...], b_ref[...])
        o_ref[...] = (jnp.dot(y, wet_ref[...], preferred_element_type=jnp.float32)
                      + be_ref[...])
    return pl.pallas_call(
        body,
        grid=(_NB,),
        in_specs=[pl.BlockSpec((_BLK, _D), lambda i: (i, 0)),
                  pl.BlockSpec((_BLK, _D), lambda i: (i + _NB, 0)),
                  pl.BlockSpec((_BLK, _D), lambda i: (i, 0)),
                  pl.BlockSpec((_BLK, _D), lambda i: (i + _NB, 0)),
                  _full((_D, _D)), _full((1, _D)), _full((1, _D)), _full((1, _D)),
                  _full((_D, _D)), _full((1, _D))],
        out_specs=pl.BlockSpec((_BLK, _D), lambda i: (i, 0)),
        out_shape=jax.ShapeDtypeStruct((_MP, _D), jnp.float32),
    )(aggp, aggp, cntp, cntp, wem_t, bem.reshape(1, _D),
      g_e.reshape(1, _D), b_e.reshape(1, _D), we_t, be.reshape(1, _D))


def _tc_post(zp, cntp, x0, w1t, w2t, bvm, g, b):
    def body(z0, z1, c0, c1, x_ref, w1_ref, w2_ref, bb_ref, g_ref, b_ref, o_ref):
        s = z0[...] + z1[...]
        c = (c0[...] + c1[...])[:, 0:1]
        z = s / jnp.clip(c, 1.0, None)
        x = x_ref[...]
        h = (jnp.dot(x, w1_ref[...], preferred_element_type=jnp.float32)
             + jnp.dot(z, w2_ref[...], preferred_element_type=jnp.float32)
             + bb_ref[...])
        o_ref[...] = _ln(h, g_ref[...], b_ref[...]) + x
    return pl.pallas_call(
        body,
        grid=(_NB,),
        in_specs=[pl.BlockSpec((_BLK, _D), lambda i: (i, 0)),
                  pl.BlockSpec((_BLK, _D), lambda i: (i + _NB, 0)),
                  pl.BlockSpec((_BLK, _D), lambda i: (i, 0)),
                  pl.BlockSpec((_BLK, _D), lambda i: (i + _NB, 0)),
                  pl.BlockSpec((_BLK, _D), lambda i: (i, 0)),
                  _full((_D, _D)), _full((_D, _D)), _full((1, _D)),
                  _full((1, _D)), _full((1, _D))],
        out_specs=pl.BlockSpec((_BLK, _D), lambda i: (i, 0)),
        out_shape=jax.ShapeDtypeStruct((_MP, _D), jnp.float32),
    )(zp, zp, cntp, cntp, x0, w1t, w2t, bvm.reshape(1, _D),
      g.reshape(1, _D), b.reshape(1, _D))


@jax.jit
def kernel(X, edge_index, Wv, bv, Wem, bem, We, be, Wvm, bvm,
           g_pre, b_pre, g_e, b_e, g_post, b_post):
    pad_idx = (_N + jnp.arange((_NCHP - _NCH) * _CHUNK, dtype=jnp.int32)
               % (_MP - _N)).reshape(_NCHP - _NCH, _CHUNK)
    v_idx = jnp.concatenate(
        [edge_index[0].astype(jnp.int32).reshape(_NCH, _CHUNK), pad_idx])
    e_idx = jnp.concatenate(
        [edge_index[1].astype(jnp.int32).reshape(_NCH, _CHUNK), pad_idx])
    x_pad = jnp.pad(X, ((0, _MP - _N), (0, 0)))
    zrow = jnp.zeros((_MP, _D), jnp.float32)
    ones = jnp.ones((_CHUNK, _D), jnp.float32)

    Xv = _tc_pre(x_pad, g_pre, b_pre, Wv.T, bv)
    aggp, cep = _sc_agg()(Xv, v_idx, e_idx, zrow, ones)
    Y = _tc_edge(aggp, cep, Wem.T, bem, g_e, b_e, We.T, be)
    zp, cvp = _sc_agg()(Y, e_idx, v_idx, zrow, ones)
    out = _tc_post(zp, cvp, x_pad, Wvm[:, :_D].T, Wvm[:, _D:].T, bvm,
                   g_post, b_post)
    return out[:_N]


# speedup vs baseline: 2.4073x; 1.0313x over previous
import jax
import jax.numpy as jnp
from jax import lax
from jax.experimental import pallas as pl
from jax.experimental.pallas import tpu as pltpu
from jax.experimental.pallas import tpu_sc as plsc

_N = 10000
_E = 320000
_D = 128
_EPS = 1e-5

_MP = 10240
_CHUNK = 128
_NCH = _E // _CHUNK
_NCHP = 2560
_WCH = _NCHP // 32
_DUMP = 10232
_RPT = _MP // 16
_CW = 16


_MESH = plsc.VectorSubcoreMesh(core_axis_name="c", subcore_axis_name="s")


def _sc_agg():
    def body(src, gidx, sidx, zrow, ones_h,
             agg_o, cnt_o, gbuf, sbuf, rows0, rows1, gsem, ssem, acc):
        cid = lax.axis_index("c")
        sid = lax.axis_index("s")
        wid = sid * 2 + cid
        base = wid * _WCH
        r0 = sid * _RPT
        half = _WCH // 2

        def zero_acc():
            pltpu.sync_copy(zrow.at[pl.ds(r0, _RPT)], acc.at[pl.ds(r0, _RPT)])

        pltpu.sync_copy(ones_h, rows0)
        zero_acc()
        plsc.subcore_barrier()

        def cfire(j, carry):
            pltpu.async_copy(rows0, acc.at[sbuf.at[carry + j]], ssem,
                             add=True)
            return carry

        for h in range(2):
            pltpu.sync_copy(sidx.at[pl.ds(base + h * half, half)], sbuf)
            for b in range(half // 8):
                lax.fori_loop(0, 8, cfire, b * 8)
                for _ in range(8):
                    pltpu.make_async_copy(zrow.at[pl.ds(0, _CHUNK)], rows1,
                                          ssem).wait()
        plsc.subcore_barrier()
        off = cid * _MP + r0
        pltpu.sync_copy(acc.at[pl.ds(r0, _RPT)], cnt_o.at[pl.ds(off, _RPT)])
        pltpu.sync_copy(gidx.at[pl.ds(base, _WCH)], gbuf)
        plsc.subcore_barrier()


        def wait_g(buf):
            pltpu.make_async_copy(src.at[gbuf.at[0]], buf, gsem).wait()

        def wait_s():
            pltpu.make_async_copy(zrow.at[pl.ds(0, _CHUNK)], rows1,
                                  ssem).wait()

        def scat(local_j, buf):
            pltpu.async_copy(buf, acc.at[sbuf.at[local_j]], ssem, add=True)

        def bstep(g, h):
            j0 = h * half + 2 * g
            hi = h * half + half - 1
            wait_g(rows0)
            scat(2 * g, rows0)
            wait_s()
            pltpu.async_copy(src.at[gbuf.at[j0 + 1]], rows1, gsem)
            wait_g(rows1)
            scat(2 * g + 1, rows1)
            wait_s()
            j2 = jnp.minimum(j0 + 2, hi)
            pltpu.async_copy(src.at[gbuf.at[j2]], rows0, gsem)
            return h

        for h in range(2):
            c0 = h * half
            pltpu.sync_copy(sidx.at[pl.ds(base + c0, half)], sbuf)
            pltpu.async_copy(src.at[gbuf.at[c0]], rows0, gsem)
            wait_g(rows0)
            scat(0, rows0)
            pltpu.async_copy(src.at[gbuf.at[c0 + 1]], rows1, gsem)
            wait_g(rows1)
            scat(1, rows1)
            wait_s()
            pltpu.async_copy(src.at[gbuf.at[c0 + 2]], rows0, gsem)
            lax.fori_loop(1, half // 2, bstep, h)
            wait_s()
            wait_g(rows0)
        plsc.subcore_barrier()
        pltpu.sync_copy(acc.at[pl.ds(r0, _RPT)], agg_o.at[pl.ds(off, _RPT)])

    return pl.kernel(
        body,
        out_type=[jax.ShapeDtypeStruct((2 * _MP, _D), jnp.float32),
                  jax.ShapeDtypeStruct((2 * _MP, _D), jnp.float32)],
        mesh=_MESH,
        scratch_types=[
            pltpu.VMEM((_WCH, _CHUNK), jnp.int32),
            pltpu.VMEM((_WCH // 2, _CHUNK), jnp.int32),
            pltpu.VMEM((_CHUNK, _D), jnp.float32),
            pltpu.VMEM((_CHUNK, _D), jnp.float32),
            pltpu.SemaphoreType.DMA,
            pltpu.SemaphoreType.DMA,
            pltpu.VMEM_SHARED((_MP, _D), jnp.float32),
        ])


_BLK = 640
_NB = _MP // _BLK


def _full(shape):
    return pl.BlockSpec(shape, lambda i: tuple(0 for _ in shape))


def _ln(x, g, b):
    mu = jnp.mean(x, axis=1, keepdims=True)
    xc = x - mu
    var = jnp.mean(xc * xc, axis=1, keepdims=True)
    return xc * lax.rsqrt(var + _EPS) * g + b


def _tc_pre(x, g, b, wt, bias):
    def body(x_ref, g_ref, b_ref, wt_ref, bb_ref, o_ref):
        xn = _ln(x_ref[...], g_ref[...], b_ref[...])
        o_ref[...] = (jnp.dot(xn, wt_ref[...], preferred_element_type=jnp.float32)
                      + bb_ref[...])
    return pl.pallas_call(
        body,
        grid=(_NB,),
        in_specs=[pl.BlockSpec((_BLK, _D), lambda i: (i, 0)),
                  _full((1, _D)), _full((1, _D)), _full((_D, _D)), _full((1, _D))],
        out_specs=pl.BlockSpec((_BLK, _D), lambda i: (i, 0)),
        out_shape=jax.ShapeDtypeStruct((_MP, _D), jnp.float32),
    )(x, g.reshape(1, _D), b.reshape(1, _D), wt, bias.reshape(1, _D))


def _tc_edge(aggp, cntp, wem_t, bem, g_e, b_e, we_t, be):
    def body(a0, a1, c0, c1, wem_ref, bem_ref, g_ref, b_ref, wet_ref, be_ref, o_ref):
        cf = c0[...] + c1[...]
        s = a0[...] + a1[...] - cf
        c = cf[:, 0:1]
        ya = s / jnp.clip(c, 1.0, None)
        y = (jnp.dot(ya, wem_ref[...], preferred_element_type=jnp.float32)
             + bem_ref[...])
        y = _ln(y, g_ref[...], b_ref[...])
        o_ref[...] = (jnp.dot(y, wet_ref[...], preferred_element_type=jnp.float32)
                      + be_ref[...])
    return pl.pallas_call(
        body,
        grid=(_NB,),
        in_specs=[pl.BlockSpec((_BLK, _D), lambda i: (i, 0)),
                  pl.BlockSpec((_BLK, _D), lambda i: (i + _NB, 0)),
                  pl.BlockSpec((_BLK, _D), lambda i: (i, 0)),
                  pl.BlockSpec((_BLK, _D), lambda i: (i + _NB, 0)),
                  _full((_D, _D)), _full((1, _D)), _full((1, _D)), _full((1, _D)),
                  _full((_D, _D)), _full((1, _D))],
        out_specs=pl.BlockSpec((_BLK, _D), lambda i: (i, 0)),
        out_shape=jax.ShapeDtypeStruct((_MP, _D), jnp.float32),
    )(aggp, aggp, cntp, cntp, wem_t, bem.reshape(1, _D),
      g_e.reshape(1, _D), b_e.reshape(1, _D), we_t, be.reshape(1, _D))


def _tc_post(zp, cntp, x0, w1t, w2t, bvm, g, b):
    def body(z0, z1, c0, c1, x_ref, w1_ref, w2_ref, bb_ref, g_ref, b_ref, o_ref):
        cf = c0[...] + c1[...]
        s = z0[...] + z1[...] - cf
        c = cf[:, 0:1]
        z = s / jnp.clip(c, 1.0, None)
        x = x_ref[...]
        h = (jnp.dot(x, w1_ref[...], preferred_element_type=jnp.float32)
             + jnp.dot(z, w2_ref[...], preferred_element_type=jnp.float32)
             + bb_ref[...])
        o_ref[...] = _ln(h, g_ref[...], b_ref[...]) + x
    return pl.pallas_call(
        body,
        grid=(_NB,),
        in_specs=[pl.BlockSpec((_BLK, _D), lambda i: (i, 0)),
                  pl.BlockSpec((_BLK, _D), lambda i: (i + _NB, 0)),
                  pl.BlockSpec((_BLK, _D), lambda i: (i, 0)),
                  pl.BlockSpec((_BLK, _D), lambda i: (i + _NB, 0)),
                  pl.BlockSpec((_BLK, _D), lambda i: (i, 0)),
                  _full((_D, _D)), _full((_D, _D)), _full((1, _D)),
                  _full((1, _D)), _full((1, _D))],
        out_specs=pl.BlockSpec((_BLK, _D), lambda i: (i, 0)),
        out_shape=jax.ShapeDtypeStruct((_MP, _D), jnp.float32),
    )(zp, zp, cntp, cntp, x0, w1t, w2t, bvm.reshape(1, _D),
      g.reshape(1, _D), b.reshape(1, _D))


@jax.jit
def kernel(X, edge_index, Wv, bv, Wem, bem, We, be, Wvm, bvm,
           g_pre, b_pre, g_e, b_e, g_post, b_post):
    pad_idx = (_N + jnp.arange((_NCHP - _NCH) * _CHUNK, dtype=jnp.int32)
               % (_MP - _N)).reshape(_NCHP - _NCH, _CHUNK)
    v_idx = jnp.concatenate(
        [edge_index[0].astype(jnp.int32).reshape(_NCH, _CHUNK), pad_idx])
    e_idx = jnp.concatenate(
        [edge_index[1].astype(jnp.int32).reshape(_NCH, _CHUNK), pad_idx])
    x_pad = jnp.pad(X, ((0, _MP - _N), (0, 0)))
    zrow = jnp.zeros((_MP, _D), jnp.float32)
    ones = jnp.ones((_CHUNK, _D), jnp.float32)

    Xv = _tc_pre(x_pad, g_pre, b_pre, Wv.T, bv)
    aggp, cep = _sc_agg()(Xv, v_idx, e_idx, zrow, ones)
    Y = _tc_edge(aggp, cep, Wem.T, bem, g_e, b_e, We.T, be)
    zp, cvp = _sc_agg()(Y, e_idx, v_idx, zrow, ones)
    out = _tc_post(zp, cvp, x_pad, Wvm[:, :_D].T, Wvm[:, _D:].T, bvm,
                   g_post, b_post)
    return out[:_N]


# speedup vs baseline: 2.4167x; 1.0039x over previous
import jax
import jax.numpy as jnp
from jax import lax
from jax.experimental import pallas as pl
from jax.experimental.pallas import tpu as pltpu
from jax.experimental.pallas import tpu_sc as plsc

_N = 10000
_E = 320000
_D = 128
_EPS = 1e-5

_MP = 10240
_CHUNK = 128
_NCH = _E // _CHUNK
_NCHP = 2560
_WCH = _NCHP // 32
_RPT = _MP // 16


_MESH = plsc.VectorSubcoreMesh(core_axis_name="c", subcore_axis_name="s")


def _sc_agg():
    def body(src, gidx, sidx, zrow, ones_h,
             agg_o, cnt_o, gbuf, sbuf, rows0, rows1, gsem, ssem, acc):
        cid = lax.axis_index("c")
        sid = lax.axis_index("s")
        wid = sid * 2 + cid
        base = wid * _WCH
        r0 = sid * _RPT
        half = _WCH // 2

        def zero_acc():
            pltpu.sync_copy(zrow.at[pl.ds(r0, _RPT)], acc.at[pl.ds(r0, _RPT)])

        pltpu.sync_copy(ones_h, rows0)
        zero_acc()
        plsc.subcore_barrier()

        def cfire(j, carry):
            pltpu.async_copy(rows0, acc.at[sbuf.at[carry + j]], ssem,
                             add=True)
            return carry

        for h in range(2):
            pltpu.sync_copy(sidx.at[pl.ds(base + h * half, half)], sbuf)
            for b in range(half // 8):
                lax.fori_loop(0, 8, cfire, b * 8)
                for _ in range(8):
                    pltpu.make_async_copy(zrow.at[pl.ds(0, _CHUNK)], rows1,
                                          ssem).wait()
        plsc.subcore_barrier()
        off = cid * _MP + r0
        pltpu.sync_copy(acc.at[pl.ds(r0, _RPT)], cnt_o.at[pl.ds(off, _RPT)])
        pltpu.sync_copy(gidx.at[pl.ds(base, _WCH)], gbuf)
        plsc.subcore_barrier()


        def wait_g(buf):
            pltpu.make_async_copy(src.at[gbuf.at[0]], buf, gsem).wait()

        def wait_s():
            pltpu.make_async_copy(zrow.at[pl.ds(0, _CHUNK)], rows1,
                                  ssem).wait()

        def scat(local_j, buf):
            pltpu.async_copy(buf, acc.at[sbuf.at[local_j]], ssem, add=True)

        def bstep(g, h):
            j0 = h * half + 2 * g
            hi = h * half + half - 1
            wait_g(rows0)
            scat(2 * g, rows0)
            wait_s()
            pltpu.async_copy(src.at[gbuf.at[j0 + 1]], rows1, gsem)
            wait_g(rows1)
            scat(2 * g + 1, rows1)
            wait_s()
            j2 = jnp.minimum(j0 + 2, hi)
            pltpu.async_copy(src.at[gbuf.at[j2]], rows0, gsem)
            return h

        for h in range(2):
            c0 = h * half
            pltpu.sync_copy(sidx.at[pl.ds(base + c0, half)], sbuf)
            pltpu.async_copy(src.at[gbuf.at[c0]], rows0, gsem)
            wait_g(rows0)
            scat(0, rows0)
            pltpu.async_copy(src.at[gbuf.at[c0 + 1]], rows1, gsem)
            wait_g(rows1)
            scat(1, rows1)
            wait_s()
            pltpu.async_copy(src.at[gbuf.at[c0 + 2]], rows0, gsem)
            lax.fori_loop(1, half // 2, bstep, h)
            wait_s()
            wait_g(rows0)
        plsc.subcore_barrier()
        pltpu.sync_copy(acc.at[pl.ds(r0, _RPT)], agg_o.at[pl.ds(off, _RPT)])

    return pl.kernel(
        body,
        out_type=[jax.ShapeDtypeStruct((2 * _MP, _D), jnp.float32),
                  jax.ShapeDtypeStruct((2 * _MP, _D), jnp.float32)],
        mesh=_MESH,
        scratch_types=[
            pltpu.VMEM((_WCH, _CHUNK), jnp.int32),
            pltpu.VMEM((_WCH // 2, _CHUNK), jnp.int32),
            pltpu.VMEM((_CHUNK, _D), jnp.float32),
            pltpu.VMEM((_CHUNK, _D), jnp.float32),
            pltpu.SemaphoreType.DMA,
            pltpu.SemaphoreType.DMA,
            pltpu.VMEM_SHARED((_MP, _D), jnp.float32),
        ])


_BLK = 640
_NB = _MP // _BLK


def _full(shape):
    return pl.BlockSpec(shape, lambda i: tuple(0 for _ in shape))


def _ln(x, g, b):
    mu = jnp.mean(x, axis=1, keepdims=True)
    xc = x - mu
    var = jnp.mean(xc * xc, axis=1, keepdims=True)
    return xc * lax.rsqrt(var + _EPS) * g + b


def _tc_pre(x, g, b, wt, bias):
    def body(x_ref, g_ref, b_ref, wt_ref, bb_ref, o_ref):
        xn = _ln(x_ref[...], g_ref[...], b_ref[...])
        o_ref[...] = (jnp.dot(xn, wt_ref[...], preferred_element_type=jnp.float32)
                      + bb_ref[...])
    return pl.pallas_call(
        body,
        grid=(_NB,),
        in_specs=[pl.BlockSpec((_BLK, _D), lambda i: (i, 0)),
                  _full((1, _D)), _full((1, _D)), _full((_D, _D)), _full((1, _D))],
        out_specs=pl.BlockSpec((_BLK, _D), lambda i: (i, 0)),
        out_shape=jax.ShapeDtypeStruct((_MP, _D), jnp.float32),
    )(x, g.reshape(1, _D), b.reshape(1, _D), wt, bias.reshape(1, _D))


def _tc_edge(aggp, cntp, wem_t, bem, g_e, b_e, we_t, be):
    def body(a0, a1, c0, c1, wem_ref, bem_ref, g_ref, b_ref, wet_ref, be_ref, o_ref):
        cf = c0[...] + c1[...]
        s = a0[...] + a1[...] - cf
        c = cf[:, 0:1]
        ya = s / jnp.clip(c, 1.0, None)
        y = (jnp.dot(ya, wem_ref[...], preferred_element_type=jnp.float32)
             + bem_ref[...])
        y = _ln(y, g_ref[...], b_ref[...])
        o_ref[...] = (jnp.dot(y, wet_ref[...], preferred_element_type=jnp.float32)
                      + be_ref[...])
    return pl.pallas_call(
        body,
        grid=(_NB,),
        in_specs=[pl.BlockSpec((_BLK, _D), lambda i: (i, 0)),
                  pl.BlockSpec((_BLK, _D), lambda i: (i + _NB, 0)),
                  pl.BlockSpec((_BLK, _D), lambda i: (i, 0)),
                  pl.BlockSpec((_BLK, _D), lambda i: (i + _NB, 0)),
                  _full((_D, _D)), _full((1, _D)), _full((1, _D)), _full((1, _D)),
                  _full((_D, _D)), _full((1, _D))],
        out_specs=pl.BlockSpec((_BLK, _D), lambda i: (i, 0)),
        out_shape=jax.ShapeDtypeStruct((_MP, _D), jnp.float32),
    )(aggp, aggp, cntp, cntp, wem_t, bem.reshape(1, _D),
      g_e.reshape(1, _D), b_e.reshape(1, _D), we_t, be.reshape(1, _D))


def _tc_post(zp, cntp, x0, w1t, w2t, bvm, g, b):
    def body(z0, z1, c0, c1, x_ref, w1_ref, w2_ref, bb_ref, g_ref, b_ref, o_ref):
        cf = c0[...] + c1[...]
        s = z0[...] + z1[...] - cf
        c = cf[:, 0:1]
        z = s / jnp.clip(c, 1.0, None)
        x = x_ref[...]
        h = (jnp.dot(x, w1_ref[...], preferred_element_type=jnp.float32)
             + jnp.dot(z, w2_ref[...], preferred_element_type=jnp.float32)
             + bb_ref[...])
        o_ref[...] = _ln(h, g_ref[...], b_ref[...]) + x
    return pl.pallas_call(
        body,
        grid=(_NB,),
        in_specs=[pl.BlockSpec((_BLK, _D), lambda i: (i, 0)),
                  pl.BlockSpec((_BLK, _D), lambda i: (i + _NB, 0)),
                  pl.BlockSpec((_BLK, _D), lambda i: (i, 0)),
                  pl.BlockSpec((_BLK, _D), lambda i: (i + _NB, 0)),
                  pl.BlockSpec((_BLK, _D), lambda i: (i, 0)),
                  _full((_D, _D)), _full((_D, _D)), _full((1, _D)),
                  _full((1, _D)), _full((1, _D))],
        out_specs=pl.BlockSpec((_BLK, _D), lambda i: (i, 0)),
        out_shape=jax.ShapeDtypeStruct((_MP, _D), jnp.float32),
    )(zp, zp, cntp, cntp, x0, w1t, w2t, bvm.reshape(1, _D),
      g.reshape(1, _D), b.reshape(1, _D))


@jax.jit
def kernel(X, edge_index, Wv, bv, Wem, bem, We, be, Wvm, bvm,
           g_pre, b_pre, g_e, b_e, g_post, b_post):
    pad_idx = (_N + jnp.arange((_NCHP - _NCH) * _CHUNK, dtype=jnp.int32)
               % (_MP - _N)).reshape(_NCHP - _NCH, _CHUNK)
    v_idx = jnp.concatenate(
        [edge_index[0].astype(jnp.int32).reshape(_NCH, _CHUNK), pad_idx])
    e_idx = jnp.concatenate(
        [edge_index[1].astype(jnp.int32).reshape(_NCH, _CHUNK), pad_idx])
    x_pad = jnp.pad(X, ((0, _MP - _N), (0, 0)))
    zrow = jnp.zeros((_MP, _D), jnp.float32)
    ones = jnp.ones((_CHUNK, _D), jnp.float32)

    Xv = _tc_pre(x_pad, g_pre, b_pre, Wv.T, bv)
    aggp, cep = _sc_agg()(Xv, v_idx, e_idx, zrow, ones)
    Y = _tc_edge(aggp, cep, Wem.T, bem, g_e, b_e, We.T, be)
    zp, cvp = _sc_agg()(Y, e_idx, v_idx, zrow, ones)
    out = _tc_post(zp, cvp, x_pad, Wvm[:, :_D].T, Wvm[:, _D:].T, bvm,
                   g_post, b_post)
    return out[:_N]
